# baseline (speedup 1.0000x reference)
import jax
import jax.numpy as jnp
from jax.experimental import pallas as pl

_BM = 2048


def _split_cat(x):
    hi = x.astype(jnp.bfloat16)
    lo = (x - hi.astype(jnp.float32)).astype(jnp.bfloat16)
    return jnp.concatenate([hi, lo], axis=1)


def _ffn_block(obs_ref, w1_ref, w2_ref, b1_ref, b2_ref, out_ref):
    n_hid = b1_ref.shape[1]
    n_out = b2_ref.shape[1]
    a1 = _split_cat(obs_ref[...])
    r1 = jnp.dot(a1, w1_ref[...], preferred_element_type=jnp.float32)
    h = jnp.tanh(r1[:, :n_hid] + r1[:, n_hid:] + b1_ref[...])
    a2 = _split_cat(h)
    r2 = jnp.dot(a2, w2_ref[...], preferred_element_type=jnp.float32)
    out_ref[...] = jnp.tanh(r2[:, :n_out] + r2[:, n_out:] + b2_ref[...])


def _pack_weights(w):
    hi = w.astype(jnp.bfloat16)
    lo = (w - hi.astype(jnp.float32)).astype(jnp.bfloat16)
    half = jnp.concatenate([hi, lo], axis=1)
    return jnp.concatenate([half, half], axis=0)


def kernel(obs, W1, W2, b1, b2):
    if obs.ndim == 1:
        obs = obs[None, :]
    batch, n_in = obs.shape
    n_hid = W1.shape[1]
    n_out = W2.shape[1]
    w1p = _pack_weights(W1)
    w2p = _pack_weights(W2)
    bm = min(_BM, batch)
    grid = (pl.cdiv(batch, bm),)
    rep = lambda i: (0, 0)
    return pl.pallas_call(
        _ffn_block,
        grid=grid,
        in_specs=[
            pl.BlockSpec((bm, n_in), lambda i: (i, 0)),
            pl.BlockSpec((2 * n_in, 2 * n_hid), rep),
            pl.BlockSpec((2 * n_hid, 2 * n_out), rep),
            pl.BlockSpec((1, n_hid), rep),
            pl.BlockSpec((1, n_out), rep),
        ],
        out_specs=pl.BlockSpec((bm, n_out), lambda i: (i, 0)),
        out_shape=jax.ShapeDtypeStruct((batch, n_out), jnp.float32),
    )(obs, w1p, w2p, b1[None, :], b2[None, :])


# speedup vs baseline: 1.1269x; 1.1269x over previous
import jax
import jax.numpy as jnp
from jax.experimental import pallas as pl

_BM = 8192


def _split_cat(x):
    hi = x.astype(jnp.bfloat16)
    lo = (x - hi.astype(jnp.float32)).astype(jnp.bfloat16)
    return jnp.concatenate([hi, lo], axis=1)


def _ffn_block(obs_ref, w1_ref, w2_ref, b1_ref, b2_ref, out_ref):
    n_hid = b1_ref.shape[1]
    n_out = b2_ref.shape[1]
    a1 = _split_cat(obs_ref[...])
    r1 = jnp.dot(a1, w1_ref[...], preferred_element_type=jnp.float32)
    h = jnp.tanh(r1[:, :n_hid] + r1[:, n_hid:] + b1_ref[...])
    a2 = _split_cat(h)
    r2 = jnp.dot(a2, w2_ref[...], preferred_element_type=jnp.float32)
    out_ref[...] = jnp.tanh(r2[:, :n_out] + r2[:, n_out:] + b2_ref[...])


def _pack_weights(w):
    hi = w.astype(jnp.bfloat16)
    lo = (w - hi.astype(jnp.float32)).astype(jnp.bfloat16)
    half = jnp.concatenate([hi, lo], axis=1)
    return jnp.concatenate([half, half], axis=0)


def kernel(obs, W1, W2, b1, b2):
    if obs.ndim == 1:
        obs = obs[None, :]
    batch, n_in = obs.shape
    n_hid = W1.shape[1]
    n_out = W2.shape[1]
    w1p = _pack_weights(W1)
    w2p = _pack_weights(W2)
    bm = min(_BM, batch)
    grid = (pl.cdiv(batch, bm),)
    rep = lambda i: (0, 0)
    return pl.pallas_call(
        _ffn_block,
        grid=grid,
        in_specs=[
            pl.BlockSpec((bm, n_in), lambda i: (i, 0)),
            pl.BlockSpec((2 * n_in, 2 * n_hid), rep),
            pl.BlockSpec((2 * n_hid, 2 * n_out), rep),
            pl.BlockSpec((1, n_hid), rep),
            pl.BlockSpec((1, n_out), rep),
        ],
        out_specs=pl.BlockSpec((bm, n_out), lambda i: (i, 0)),
        out_shape=jax.ShapeDtypeStruct((batch, n_out), jnp.float32),
    )(obs, w1p, w2p, b1[None, :], b2[None, :])
